# baseline (speedup 1.0000x reference)
import functools

import jax
import jax.numpy as jnp
from jax import lax
from jax.experimental import pallas as pl
from jax.experimental.pallas import tpu as pltpu
from jax.experimental.pallas import tpu_sc as plsc

VOCAB = 1000000
HID = 64
B = 16384
L = 200

NTILES = 32
RPT = B // NTILES
W0 = 104
W1 = L - W0
G = 128
NCH = RPT // G
RING = 8

_RC = 8192
_NRB = 123
VPAIR = _NRB * (_RC // 2)
VPAD = 2 * VPAIR

_mesh = plsc.VectorSubcoreMesh(core_axis_name="c", subcore_axis_name="s")


@functools.partial(
    pl.kernel,
    mesh=_mesh,
    out_type=jax.ShapeDtypeStruct((B, HID), jnp.float32),
    scratch_types=[
        pltpu.VMEM((G * L,), jnp.int32),
        pltpu.VMEM((RING, W0, HID), jnp.float32),
        pltpu.VMEM((G, HID), jnp.float32),
    ] + [pltpu.SemaphoreType.DMA] * RING,
    compiler_params=pltpu.CompilerParams(use_tc_tiling_on_sc=False),
)
def _gather_pool(ids_hbm, emb_hbm, out_hbm, idx_v, rows_v, out_v, *sems):
    wid = lax.axis_index("s") * 2 + lax.axis_index("c")
    row0 = wid * RPT

    def _win(b, r, h):
        off, ln = (0, W0) if h == 0 else (W0, W1)
        src = emb_hbm.at[idx_v.at[pl.ds(r * L + off, ln)]]
        dst = rows_v.at[b, pl.ds(0, ln)]
        return src, dst

    def _accum_window(b, ln, acc):
        def body(i, acc):
            a0, a1, a2, a3 = acc
            a0 = a0 + rows_v[b, i, pl.ds(0, 16)]
            a1 = a1 + rows_v[b, i, pl.ds(16, 16)]
            a2 = a2 + rows_v[b, i, pl.ds(32, 16)]
            a3 = a3 + rows_v[b, i, pl.ds(48, 16)]
            return (a0, a1, a2, a3)
        return lax.fori_loop(0, ln, body, acc, unroll=8)

    @pl.loop(0, NCH)
    def _chunk(c):
        base = row0 + c * G
        pltpu.sync_copy(ids_hbm.at[pl.ds(base * L, G * L)], idx_v)
        for b in range(RING):
            src, dst = _win(b, b // 2, b % 2)
            pltpu.async_copy(src, dst, sems[b])

        @pl.loop(0, 2 * G, step=RING)
        def _group(w):
            for pair in range(RING // 2):
                r = w // 2 + pair
                zero = jnp.zeros((16,), jnp.float32)
                acc = (zero, zero, zero, zero)
                for h in range(2):
                    b = pair * 2 + h
                    src, dst = _win(b, r, h)
                    pltpu.make_async_copy(src, dst, sems[b]).wait()
                    acc = _accum_window(b, W0 if h == 0 else W1, acc)

                    @pl.when(w + RING + b < 2 * G)
                    def _refire():
                        src, dst = _win(b, r + RING // 2, h)
                        pltpu.async_copy(src, dst, sems[b])
                out_v[r, pl.ds(0, 16)] = acc[0]
                out_v[r, pl.ds(16, 16)] = acc[1]
                out_v[r, pl.ds(32, 16)] = acc[2]
                out_v[r, pl.ds(48, 16)] = acc[3]

        pltpu.sync_copy(out_v, out_hbm.at[pl.ds(base, G)])


def _relayout_body(x_ref, o_ref):
    x = x_ref[...]
    ident = jnp.eye(HID, dtype=jnp.float32)
    dn = (((0,), (0,)), ((), ()))
    o_ref[:, 0:HID] = lax.dot_general(
        x[:, 0:_RC // 2], ident, dn, preferred_element_type=jnp.float32,
        precision=lax.Precision.DEFAULT)
    o_ref[:, HID:2 * HID] = lax.dot_general(
        x[:, _RC // 2:_RC], ident, dn, preferred_element_type=jnp.float32,
        precision=lax.Precision.DEFAULT)


def _relayout(emb_t):
    return pl.pallas_call(
        _relayout_body,
        grid=(_NRB,),
        in_specs=[pl.BlockSpec((HID, _RC), lambda i: (0, i))],
        out_specs=pl.BlockSpec((_RC // 2, 2 * HID), lambda i: (i, 0)),
        out_shape=jax.ShapeDtypeStruct((VPAIR, 2 * HID), jnp.float32),
    )(emb_t)


def _erf_poly(x):
    a1, a2, a3, a4, a5 = (
        0.254829592, -0.284496736, 1.421413741, -1.453152027, 1.061405429)
    p = 0.3275911
    s = jnp.sign(x)
    ax = jnp.abs(x)
    t = 1.0 / (1.0 + p * ax)
    poly = t * (a1 + t * (a2 + t * (a3 + t * (a4 + t * a5))))
    return s * (1.0 - poly * jnp.exp(-ax * ax))


def _gelu(x):
    return 0.5 * x * (1.0 + _erf_poly(x * jnp.float32(0.7071067811865476)))


def _mlp_body(x_ref, w1t_ref, b1_ref, w2t_ref, b2_ref, o_ref):
    x = x_ref[...] / jnp.float32(float(L))
    h = _gelu(jnp.dot(x, w1t_ref[...], preferred_element_type=jnp.float32,
                      precision=lax.Precision.HIGHEST) + b1_ref[...])
    o_ref[...] = _gelu(jnp.dot(h, w2t_ref[...], preferred_element_type=jnp.float32,
                               precision=lax.Precision.HIGHEST) + b2_ref[...])


_BM = 4096


def _mlp(pooled2, w1d, b1d, w2d, b2d):
    grid = (B // 2 // _BM,)
    return pl.pallas_call(
        _mlp_body,
        grid=grid,
        in_specs=[
            pl.BlockSpec((_BM, 2 * HID), lambda i: (i, 0)),
            pl.BlockSpec((2 * HID, 2 * HID), lambda i: (0, 0)),
            pl.BlockSpec((1, 2 * HID), lambda i: (0, 0)),
            pl.BlockSpec((2 * HID, 2 * HID), lambda i: (0, 0)),
            pl.BlockSpec((1, 2 * HID), lambda i: (0, 0)),
        ],
        out_specs=pl.BlockSpec((_BM, 2 * HID), lambda i: (i, 0)),
        out_shape=jax.ShapeDtypeStruct((B // 2, 2 * HID), jnp.float32),
    )(pooled2, w1d, b1d, w2d, b2d)


def _blockdiag(w):
    z = jnp.zeros((HID, HID), w.dtype)
    return jnp.concatenate([
        jnp.concatenate([w, z], axis=1),
        jnp.concatenate([z, w], axis=1)], axis=0)


def kernel(build_order_trace, emb, W1, b1, W2, b2):
    v = build_order_trace.reshape(-1).astype(jnp.int32) + 1
    r = v & (_RC - 1)
    ids_t = (v - r) + ((r & (_RC // 2 - 1)) << 1) + (r >> 12)
    table = _relayout(emb.T).reshape(VPAD, HID)
    pooled2 = _gather_pool(ids_t, table).reshape(B // 2, 2 * HID)
    b1d = jnp.concatenate([b1, b1]).reshape(1, 2 * HID)
    b2d = jnp.concatenate([b2, b2]).reshape(1, 2 * HID)
    out2 = _mlp(pooled2, _blockdiag(W1.T), b1d, _blockdiag(W2.T), b2d)
    return out2.reshape(B, HID)


# speedup vs baseline: 1.0206x; 1.0206x over previous
import functools

import jax
import jax.numpy as jnp
from jax import lax
from jax.experimental import pallas as pl
from jax.experimental.pallas import tpu as pltpu
from jax.experimental.pallas import tpu_sc as plsc

VOCAB = 1000000
HID = 64
B = 16384
L = 200

NTILES = 32
RPT = B // NTILES
W0 = 104
W1 = L - W0
G = 128
NCH = RPT // G
RING = 8

_RC = 8192
_NRB = 123
VPAIR = _NRB * (_RC // 2)
VPAD = 2 * VPAIR

_mesh = plsc.VectorSubcoreMesh(core_axis_name="c", subcore_axis_name="s")


@functools.partial(
    pl.kernel,
    mesh=_mesh,
    out_type=jax.ShapeDtypeStruct((B, HID), jnp.float32),
    scratch_types=[
        pltpu.VMEM((G * L,), jnp.int32),
        pltpu.VMEM((RING, W0, HID), jnp.float32),
        pltpu.VMEM((G, HID), jnp.float32),
    ] + [pltpu.SemaphoreType.DMA] * RING,
    compiler_params=pltpu.CompilerParams(use_tc_tiling_on_sc=False),
)
def _gather_pool(ids_hbm, emb_hbm, out_hbm, idx_v, rows_v, out_v, *sems):
    wid = lax.axis_index("s") * 2 + lax.axis_index("c")
    row0 = wid * RPT

    def _win(b, r, h):
        off, ln = (0, W0) if h == 0 else (W0, W1)
        src = emb_hbm.at[idx_v.at[pl.ds(r * L + off, ln)]]
        dst = rows_v.at[b, pl.ds(0, ln)]
        return src, dst

    def _accum_window(b, ln, acc):
        def body(i, acc):
            a0, a1, a2, a3 = acc
            a0 = a0 + rows_v[b, i, pl.ds(0, 16)]
            a1 = a1 + rows_v[b, i, pl.ds(16, 16)]
            a2 = a2 + rows_v[b, i, pl.ds(32, 16)]
            a3 = a3 + rows_v[b, i, pl.ds(48, 16)]
            return (a0, a1, a2, a3)
        return lax.fori_loop(0, ln, body, acc, unroll=8)

    @pl.loop(0, NCH)
    def _chunk(c):
        base = row0 + c * G
        pltpu.sync_copy(ids_hbm.at[pl.ds(base * L, G * L)], idx_v)
        for b in range(RING):
            src, dst = _win(b, b // 2, b % 2)
            pltpu.async_copy(src, dst, sems[b])

        @pl.loop(0, 2 * G, step=RING)
        def _group(w):
            for pair in range(RING // 2):
                r = w // 2 + pair
                zero = jnp.zeros((16,), jnp.float32)
                acc = (zero, zero, zero, zero)
                for h in range(2):
                    b = pair * 2 + h
                    src, dst = _win(b, r, h)
                    pltpu.make_async_copy(src, dst, sems[b]).wait()
                    acc = _accum_window(b, W0 if h == 0 else W1, acc)

                    @pl.when(w + RING + b < 2 * G)
                    def _refire():
                        src, dst = _win(b, r + RING // 2, h)
                        pltpu.async_copy(src, dst, sems[b])
                out_v[r, pl.ds(0, 16)] = acc[0]
                out_v[r, pl.ds(16, 16)] = acc[1]
                out_v[r, pl.ds(32, 16)] = acc[2]
                out_v[r, pl.ds(48, 16)] = acc[3]

        pltpu.sync_copy(out_v, out_hbm.at[pl.ds(base, G)])


def _relayout_body(x_ref, o_ref):
    x = x_ref[...].astype(jnp.bfloat16)
    ident = jnp.eye(HID, dtype=jnp.bfloat16)
    dn = (((0,), (0,)), ((), ()))
    o_ref[:, 0:HID] = lax.dot_general(
        x[:, 0:_RC // 2], ident, dn, preferred_element_type=jnp.float32)
    o_ref[:, HID:2 * HID] = lax.dot_general(
        x[:, _RC // 2:_RC], ident, dn, preferred_element_type=jnp.float32)


def _relayout(emb_t):
    return pl.pallas_call(
        _relayout_body,
        grid=(_NRB,),
        in_specs=[pl.BlockSpec((HID, _RC), lambda i: (0, i))],
        out_specs=pl.BlockSpec((_RC // 2, 2 * HID), lambda i: (i, 0)),
        out_shape=jax.ShapeDtypeStruct((VPAIR, 2 * HID), jnp.float32),
    )(emb_t)


def _erf_poly(x):
    a1, a2, a3, a4, a5 = (
        0.254829592, -0.284496736, 1.421413741, -1.453152027, 1.061405429)
    p = 0.3275911
    s = jnp.sign(x)
    ax = jnp.abs(x)
    t = 1.0 / (1.0 + p * ax)
    poly = t * (a1 + t * (a2 + t * (a3 + t * (a4 + t * a5))))
    return s * (1.0 - poly * jnp.exp(-ax * ax))


def _gelu(x):
    return 0.5 * x * (1.0 + _erf_poly(x * jnp.float32(0.7071067811865476)))


def _mlp_body(x_ref, w1t_ref, b1_ref, w2t_ref, b2_ref, o_ref):
    x = x_ref[...] / jnp.float32(float(L))
    h = _gelu(jnp.dot(x, w1t_ref[...], preferred_element_type=jnp.float32,
                      precision=lax.Precision.HIGHEST) + b1_ref[...])
    o_ref[...] = _gelu(jnp.dot(h, w2t_ref[...], preferred_element_type=jnp.float32,
                               precision=lax.Precision.HIGHEST) + b2_ref[...])


_BM = 4096


def _mlp(pooled_sum, w1t, b1, w2t, b2):
    grid = (B // _BM,)
    return pl.pallas_call(
        _mlp_body,
        grid=grid,
        in_specs=[
            pl.BlockSpec((_BM, HID), lambda i: (i, 0)),
            pl.BlockSpec((HID, HID), lambda i: (0, 0)),
            pl.BlockSpec((1, HID), lambda i: (0, 0)),
            pl.BlockSpec((HID, HID), lambda i: (0, 0)),
            pl.BlockSpec((1, HID), lambda i: (0, 0)),
        ],
        out_specs=pl.BlockSpec((_BM, HID), lambda i: (i, 0)),
        out_shape=jax.ShapeDtypeStruct((B, HID), jnp.float32),
    )(pooled_sum, w1t, b1, w2t, b2)


def kernel(build_order_trace, emb, W1, b1, W2, b2):
    v = build_order_trace.reshape(-1).astype(jnp.int32) + 1
    r = v & (_RC - 1)
    ids_t = (v - r) + ((r & (_RC // 2 - 1)) << 1) + (r >> 12)
    table = _relayout(emb.T).reshape(VPAD, HID)
    pooled_sum = _gather_pool(ids_t, table)
    return _mlp(pooled_sum, W1.T, b1.reshape(1, HID), W2.T, b2.reshape(1, HID))


# speedup vs baseline: 1.0572x; 1.0358x over previous
import functools

import jax
import jax.numpy as jnp
from jax import lax
from jax.experimental import pallas as pl
from jax.experimental.pallas import tpu as pltpu
from jax.experimental.pallas import tpu_sc as plsc

VOCAB = 1000000
HID = 64
B = 16384
L = 200

NTILES = 32
RPT = B // NTILES
W0 = 104
W1 = L - W0
G = 128
NCH = RPT // G
RING = 8

_RC = 8192
_NRB = 123
VPAIR = _NRB * (_RC // 2)
VPAD = 2 * VPAIR

_mesh = plsc.VectorSubcoreMesh(core_axis_name="c", subcore_axis_name="s")


@functools.partial(
    pl.kernel,
    mesh=_mesh,
    out_type=jax.ShapeDtypeStruct((B, HID), jnp.float32),
    scratch_types=[
        pltpu.VMEM((G * L,), jnp.int32),
        pltpu.VMEM((RING, W0, HID), jnp.float32),
        pltpu.VMEM((G, HID), jnp.float32),
    ] + [pltpu.SemaphoreType.DMA] * RING,
    compiler_params=pltpu.CompilerParams(use_tc_tiling_on_sc=False),
)
def _gather_pool(ids_hbm, emb_hbm, out_hbm, idx_v, rows_v, out_v, *sems):
    wid = lax.axis_index("s") * 2 + lax.axis_index("c")
    row0 = wid * RPT

    def _win(b, r, h):
        off, ln = (0, W0) if h == 0 else (W0, W1)
        src = emb_hbm.at[idx_v.at[pl.ds(r * L + off, ln)]]
        dst = rows_v.at[b, pl.ds(0, ln)]
        return src, dst

    def _accum_window(b, ln, acc):
        def body(i, acc):
            a0, a1, a2, a3 = acc
            a0 = a0 + rows_v[b, i, pl.ds(0, 16)]
            a1 = a1 + rows_v[b, i, pl.ds(16, 16)]
            a2 = a2 + rows_v[b, i, pl.ds(32, 16)]
            a3 = a3 + rows_v[b, i, pl.ds(48, 16)]
            return (a0, a1, a2, a3)
        return lax.fori_loop(0, ln, body, acc, unroll=8)

    @pl.loop(0, NCH)
    def _chunk(c):
        base = row0 + c * G
        pltpu.sync_copy(ids_hbm.at[pl.ds(base * L, G * L)], idx_v)
        for b in range(RING):
            src, dst = _win(b, b // 2, b % 2)
            pltpu.async_copy(src, dst, sems[b])

        @pl.loop(0, 2 * G, step=RING)
        def _group(w):
            for pair in range(RING // 2):
                r = w // 2 + pair
                zero = jnp.zeros((16,), jnp.float32)
                acc = (zero, zero, zero, zero)
                for h in range(2):
                    b = pair * 2 + h
                    src, dst = _win(b, r, h)
                    pltpu.make_async_copy(src, dst, sems[b]).wait()
                    acc = _accum_window(b, W0 if h == 0 else W1, acc)

                    @pl.when(w + RING + b < 2 * G)
                    def _refire():
                        src, dst = _win(b, r + RING // 2, h)
                        pltpu.async_copy(src, dst, sems[b])
                out_v[r, pl.ds(0, 16)] = acc[0]
                out_v[r, pl.ds(16, 16)] = acc[1]
                out_v[r, pl.ds(32, 16)] = acc[2]
                out_v[r, pl.ds(48, 16)] = acc[3]

        pltpu.sync_copy(out_v, out_hbm.at[pl.ds(base, G)])


def _relayout_body(x_ref, o_ref):
    x = x_ref[...].astype(jnp.bfloat16)
    ident = jnp.eye(HID, dtype=jnp.bfloat16)
    dn = (((0,), (0,)), ((), ()))
    o_ref[:, 0:HID] = lax.dot_general(
        x[:, 0:_RC // 2], ident, dn, preferred_element_type=jnp.float32)
    o_ref[:, HID:2 * HID] = lax.dot_general(
        x[:, _RC // 2:_RC], ident, dn, preferred_element_type=jnp.float32)


def _relayout(emb_t):
    return pl.pallas_call(
        _relayout_body,
        grid=(_NRB,),
        in_specs=[pl.BlockSpec((HID, _RC), lambda i: (0, i))],
        out_specs=pl.BlockSpec((_RC // 2, 2 * HID), lambda i: (i, 0)),
        out_shape=jax.ShapeDtypeStruct((VPAIR, 2 * HID), jnp.float32),
    )(emb_t)


def _erf_poly(x):
    a1, a2, a3, a4, a5 = (
        0.254829592, -0.284496736, 1.421413741, -1.453152027, 1.061405429)
    p = 0.3275911
    s = jnp.sign(x)
    ax = jnp.abs(x)
    t = 1.0 / (1.0 + p * ax)
    poly = t * (a1 + t * (a2 + t * (a3 + t * (a4 + t * a5))))
    return s * (1.0 - poly * jnp.exp(-ax * ax))


def _gelu(x):
    return 0.5 * x * (1.0 + _erf_poly(x * jnp.float32(0.7071067811865476)))


def _mlp_body(x_ref, w1t_ref, b1_ref, w2t_ref, b2_ref, o_ref):
    x = x_ref[...] / jnp.float32(float(L))
    h = _gelu(jnp.dot(x, w1t_ref[...], preferred_element_type=jnp.float32)
              + b1_ref[...])
    o_ref[...] = _gelu(jnp.dot(h, w2t_ref[...], preferred_element_type=jnp.float32)
                       + b2_ref[...])


_BM = 4096


def _mlp(pooled_sum, w1t, b1, w2t, b2):
    grid = (B // _BM,)
    return pl.pallas_call(
        _mlp_body,
        grid=grid,
        in_specs=[
            pl.BlockSpec((_BM, HID), lambda i: (i, 0)),
            pl.BlockSpec((HID, HID), lambda i: (0, 0)),
            pl.BlockSpec((1, HID), lambda i: (0, 0)),
            pl.BlockSpec((HID, HID), lambda i: (0, 0)),
            pl.BlockSpec((1, HID), lambda i: (0, 0)),
        ],
        out_specs=pl.BlockSpec((_BM, HID), lambda i: (i, 0)),
        out_shape=jax.ShapeDtypeStruct((B, HID), jnp.float32),
    )(pooled_sum, w1t, b1, w2t, b2)


def kernel(build_order_trace, emb, W1, b1, W2, b2):
    v = build_order_trace.reshape(-1).astype(jnp.int32) + 1
    r = v & (_RC - 1)
    ids_t = (v - r) + ((r & (_RC // 2 - 1)) << 1) + (r >> 12)
    table = _relayout(emb.T).reshape(VPAD, HID)
    pooled_sum = _gather_pool(ids_t, table)
    return _mlp(pooled_sum, W1.T, b1.reshape(1, HID), W2.T, b2.reshape(1, HID))


# speedup vs baseline: 1.3565x; 1.2831x over previous
import functools

import numpy as np
import jax
import jax.numpy as jnp
from jax import lax
from jax.experimental import pallas as pl
from jax.experimental.pallas import tpu as pltpu
from jax.experimental.pallas import tpu_sc as plsc

VOCAB = 1000000
HID = 64
B = 16384
L = 200

NTILES = 32
RPT = B // NTILES
W0 = 104
W1W = L - W0
G = 128
NCH = RPT // G
RING = 8

PACK = HID // 2

_RC = 16384
_QC = _RC // 4
_NRB = 62
VQUAD = _NRB * _QC
VPAD = 4 * VQUAD

_PERM = np.concatenate([
    np.arange(0, 32, 2), np.arange(1, 32, 2),
    np.arange(32, 64, 2), np.arange(33, 64, 2)])
_I_EVEN = np.asarray(np.eye(HID)[:, 0::2], np.float32)
_I_ODD = np.asarray(np.eye(HID)[:, 1::2], np.float32)

_mesh = plsc.VectorSubcoreMesh(core_axis_name="c", subcore_axis_name="s")


@functools.partial(
    pl.kernel,
    mesh=_mesh,
    out_type=jax.ShapeDtypeStruct((B, HID), jnp.float32),
    scratch_types=[
        pltpu.VMEM((G * L,), jnp.int32),
        pltpu.VMEM((RING, W0, PACK), jnp.int32),
        pltpu.VMEM((G, HID), jnp.float32),
    ] + [pltpu.SemaphoreType.DMA] * RING,
    compiler_params=pltpu.CompilerParams(use_tc_tiling_on_sc=False,
                                         needs_layout_passes=False),
)
def _gather_pool(ids_hbm, emb_hbm, out_hbm, idx_v, rows_v, out_v, *sems):
    wid = lax.axis_index("s") * 2 + lax.axis_index("c")
    row0 = wid * RPT

    def _win(b, r, h):
        off, ln = (0, W0) if h == 0 else (W0, W1W)
        src = emb_hbm.at[idx_v.at[pl.ds(r * L + off, ln)]]
        dst = rows_v.at[b, pl.ds(0, ln)]
        return src, dst

    def _accum_window(b, ln, acc):
        def body(i, acc):
            a0, a1, a2, a3 = acc
            p0 = plsc.bitcast(rows_v[b, i, pl.ds(0, 16)], jnp.bfloat16)
            p1 = plsc.bitcast(rows_v[b, i, pl.ds(16, 16)], jnp.bfloat16)
            e0, o0 = plsc.unpack(p0, format=plsc.PackFormat.INTERLEAVED,
                                 preferred_element_type=jnp.float32)
            e1, o1 = plsc.unpack(p1, format=plsc.PackFormat.INTERLEAVED,
                                 preferred_element_type=jnp.float32)
            return (a0 + e0, a1 + o0, a2 + e1, a3 + o1)
        return lax.fori_loop(0, ln, body, acc, unroll=8)

    @pl.loop(0, NCH)
    def _chunk(c):
        base = row0 + c * G
        pltpu.sync_copy(ids_hbm.at[pl.ds(base * L, G * L)], idx_v)
        for b in range(RING):
            src, dst = _win(b, b // 2, b % 2)
            pltpu.async_copy(src, dst, sems[b])

        @pl.loop(0, 2 * G, step=RING)
        def _group(w):
            for pair in range(RING // 2):
                r = w // 2 + pair
                zero = jnp.zeros((16,), jnp.float32)
                acc = (zero, zero, zero, zero)
                for h in range(2):
                    b = pair * 2 + h
                    src, dst = _win(b, r, h)
                    pltpu.make_async_copy(src, dst, sems[b]).wait()
                    acc = _accum_window(b, W0 if h == 0 else W1W, acc)

                    @pl.when(w + RING + b < 2 * G)
                    def _refire():
                        src, dst = _win(b, r + RING // 2, h)
                        pltpu.async_copy(src, dst, sems[b])
                out_v[r, pl.ds(0, 16)] = acc[0]
                out_v[r, pl.ds(16, 16)] = acc[1]
                out_v[r, pl.ds(32, 16)] = acc[2]
                out_v[r, pl.ds(48, 16)] = acc[3]

        pltpu.sync_copy(out_v, out_hbm.at[pl.ds(base, G)])


def _pack_bf16(even_f32, odd_f32):
    ie = lax.bitcast_convert_type(even_f32, jnp.int32) + jnp.int32(0x8000)
    io = lax.bitcast_convert_type(odd_f32, jnp.int32) + jnp.int32(0x8000)
    lo = lax.shift_right_logical(ie, 16)
    hi = jnp.bitwise_and(io, jnp.int32(-65536))
    return jnp.bitwise_or(hi, lo)


def _relayout_body(x_ref, ie_ref, io_ref, o_ref):
    x = x_ref[...].astype(jnp.bfloat16)
    ie = ie_ref[...]
    io = io_ref[...]
    dn = (((0,), (0,)), ((), ()))
    for q in range(4):
        xq = x[:, q * _QC:(q + 1) * _QC]
        ye = lax.dot_general(xq, ie, dn, preferred_element_type=jnp.float32)
        yo = lax.dot_general(xq, io, dn, preferred_element_type=jnp.float32)
        o_ref[:, q * PACK:(q + 1) * PACK] = _pack_bf16(ye, yo)


def _relayout(emb_t):
    return pl.pallas_call(
        _relayout_body,
        grid=(_NRB,),
        in_specs=[
            pl.BlockSpec((HID, _RC), lambda i: (0, i)),
            pl.BlockSpec((HID, PACK), lambda i: (0, 0)),
            pl.BlockSpec((HID, PACK), lambda i: (0, 0)),
        ],
        out_specs=pl.BlockSpec((_QC, 4 * PACK), lambda i: (i, 0)),
        out_shape=jax.ShapeDtypeStruct((VQUAD, 4 * PACK), jnp.int32),
    )(emb_t, jnp.asarray(_I_EVEN, jnp.bfloat16), jnp.asarray(_I_ODD, jnp.bfloat16))


def _erf_poly(x):
    a1, a2, a3, a4, a5 = (
        0.254829592, -0.284496736, 1.421413741, -1.453152027, 1.061405429)
    p = 0.3275911
    s = jnp.sign(x)
    ax = jnp.abs(x)
    t = 1.0 / (1.0 + p * ax)
    poly = t * (a1 + t * (a2 + t * (a3 + t * (a4 + t * a5))))
    return s * (1.0 - poly * jnp.exp(-ax * ax))


def _gelu(x):
    return 0.5 * x * (1.0 + _erf_poly(x * jnp.float32(0.7071067811865476)))


def _mlp_body(x_ref, w1t_ref, b1_ref, w2t_ref, b2_ref, o_ref):
    x = x_ref[...] / jnp.float32(float(L))
    h = _gelu(jnp.dot(x, w1t_ref[...], preferred_element_type=jnp.float32)
              + b1_ref[...])
    o_ref[...] = _gelu(jnp.dot(h, w2t_ref[...], preferred_element_type=jnp.float32)
                       + b2_ref[...])


_BM = 4096


def _mlp(pooled_sum, w1t, b1, w2t, b2):
    grid = (B // _BM,)
    return pl.pallas_call(
        _mlp_body,
        grid=grid,
        in_specs=[
            pl.BlockSpec((_BM, HID), lambda i: (i, 0)),
            pl.BlockSpec((HID, HID), lambda i: (0, 0)),
            pl.BlockSpec((1, HID), lambda i: (0, 0)),
            pl.BlockSpec((HID, HID), lambda i: (0, 0)),
            pl.BlockSpec((1, HID), lambda i: (0, 0)),
        ],
        out_specs=pl.BlockSpec((_BM, HID), lambda i: (i, 0)),
        out_shape=jax.ShapeDtypeStruct((B, HID), jnp.float32),
    )(pooled_sum, w1t, b1, w2t, b2)


def kernel(build_order_trace, emb, W1, b1, W2, b2):
    v = build_order_trace.reshape(-1).astype(jnp.int32) + 1
    r = v & (_RC - 1)
    ids_t = (v - r) + ((r & (_QC - 1)) << 2) + (r >> 12)
    table = _relayout(emb.T).reshape(VPAD, PACK)
    pooled_sum = _gather_pool(ids_t, table)
    w1tp = W1.T[_PERM, :]
    return _mlp(pooled_sum, w1tp, b1.reshape(1, HID), W2.T, b2.reshape(1, HID))


# speedup vs baseline: 1.3845x; 1.0206x over previous
import functools

import numpy as np
import jax
import jax.numpy as jnp
from jax import lax
from jax.experimental import pallas as pl
from jax.experimental.pallas import tpu as pltpu
from jax.experimental.pallas import tpu_sc as plsc

VOCAB = 1000000
HID = 64
B = 16384
L = 200

NTILES = 32
RPT = B // NTILES
W0 = 104
W1W = L - W0
G = 128
NCH = RPT // G
RING = 8

PACK = HID // 2

_RC = 32768
_QC = _RC // 4
_NRB = 31
VQUAD = _NRB * _QC
VPAD = 4 * VQUAD

_PERM = np.concatenate([
    np.arange(0, 32, 2), np.arange(1, 32, 2),
    np.arange(32, 64, 2), np.arange(33, 64, 2)])
_I_EVEN = np.asarray(np.eye(HID)[:, 0::2], np.float32)
_I_ODD = np.asarray(np.eye(HID)[:, 1::2], np.float32)

_mesh = plsc.VectorSubcoreMesh(core_axis_name="c", subcore_axis_name="s")


@functools.partial(
    pl.kernel,
    mesh=_mesh,
    out_type=jax.ShapeDtypeStruct((B, HID), jnp.float32),
    scratch_types=[
        pltpu.VMEM((G * L,), jnp.int32),
        pltpu.VMEM((RING, W0, PACK), jnp.int32),
        pltpu.VMEM((G, HID), jnp.float32),
    ] + [pltpu.SemaphoreType.DMA] * RING,
    compiler_params=pltpu.CompilerParams(use_tc_tiling_on_sc=False,
                                         needs_layout_passes=False),
)
def _gather_pool(ids_hbm, emb_hbm, out_hbm, idx_v, rows_v, out_v, *sems):
    wid = lax.axis_index("s") * 2 + lax.axis_index("c")
    row0 = wid * RPT

    def _win(b, r, h):
        off, ln = (0, W0) if h == 0 else (W0, W1W)
        src = emb_hbm.at[idx_v.at[pl.ds(r * L + off, ln)]]
        dst = rows_v.at[b, pl.ds(0, ln)]
        return src, dst

    def _accum_window(b, ln, acc):
        def body(i, acc):
            a0, a1, a2, a3 = acc
            p0 = plsc.bitcast(rows_v[b, i, pl.ds(0, 16)], jnp.bfloat16)
            p1 = plsc.bitcast(rows_v[b, i, pl.ds(16, 16)], jnp.bfloat16)
            e0, o0 = plsc.unpack(p0, format=plsc.PackFormat.INTERLEAVED,
                                 preferred_element_type=jnp.float32)
            e1, o1 = plsc.unpack(p1, format=plsc.PackFormat.INTERLEAVED,
                                 preferred_element_type=jnp.float32)
            return (a0 + e0, a1 + o0, a2 + e1, a3 + o1)
        return lax.fori_loop(0, ln, body, acc, unroll=8)

    @pl.loop(0, NCH)
    def _chunk(c):
        base = row0 + c * G
        pltpu.sync_copy(ids_hbm.at[pl.ds(base * L, G * L)], idx_v)
        for b in range(RING):
            src, dst = _win(b, b // 2, b % 2)
            pltpu.async_copy(src, dst, sems[b])

        @pl.loop(0, 2 * G, step=RING)
        def _group(w):
            for pair in range(RING // 2):
                r = w // 2 + pair
                zero = jnp.zeros((16,), jnp.float32)
                acc = (zero, zero, zero, zero)
                for h in range(2):
                    b = pair * 2 + h
                    src, dst = _win(b, r, h)
                    pltpu.make_async_copy(src, dst, sems[b]).wait()
                    acc = _accum_window(b, W0 if h == 0 else W1W, acc)

                    @pl.when(w + RING + b < 2 * G)
                    def _refire():
                        src, dst = _win(b, r + RING // 2, h)
                        pltpu.async_copy(src, dst, sems[b])
                out_v[r, pl.ds(0, 16)] = acc[0]
                out_v[r, pl.ds(16, 16)] = acc[1]
                out_v[r, pl.ds(32, 16)] = acc[2]
                out_v[r, pl.ds(48, 16)] = acc[3]

        pltpu.sync_copy(out_v, out_hbm.at[pl.ds(base, G)])


def _pack_bf16(even_f32, odd_f32):
    ie = lax.bitcast_convert_type(even_f32, jnp.int32) + jnp.int32(0x8000)
    io = lax.bitcast_convert_type(odd_f32, jnp.int32) + jnp.int32(0x8000)
    lo = lax.shift_right_logical(ie, 16)
    hi = jnp.bitwise_and(io, jnp.int32(-65536))
    return jnp.bitwise_or(hi, lo)


def _relayout_body(x_ref, ie_ref, io_ref, o_ref):
    x = x_ref[...].astype(jnp.bfloat16)
    ie = ie_ref[...]
    io = io_ref[...]
    dn = (((0,), (0,)), ((), ()))
    for q in range(4):
        xq = x[:, q * _QC:(q + 1) * _QC]
        ye = lax.dot_general(xq, ie, dn, preferred_element_type=jnp.float32)
        yo = lax.dot_general(xq, io, dn, preferred_element_type=jnp.float32)
        o_ref[:, q * PACK:(q + 1) * PACK] = _pack_bf16(ye, yo)


def _relayout(emb_t):
    return pl.pallas_call(
        _relayout_body,
        grid=(_NRB,),
        in_specs=[
            pl.BlockSpec((HID, _RC), lambda i: (0, i)),
            pl.BlockSpec((HID, PACK), lambda i: (0, 0)),
            pl.BlockSpec((HID, PACK), lambda i: (0, 0)),
        ],
        out_specs=pl.BlockSpec((_QC, 4 * PACK), lambda i: (i, 0)),
        out_shape=jax.ShapeDtypeStruct((VQUAD, 4 * PACK), jnp.int32),
    )(emb_t, jnp.asarray(_I_EVEN, jnp.bfloat16), jnp.asarray(_I_ODD, jnp.bfloat16))


def _erf_poly(x):
    a1, a2, a3, a4, a5 = (
        0.254829592, -0.284496736, 1.421413741, -1.453152027, 1.061405429)
    p = 0.3275911
    s = jnp.sign(x)
    ax = jnp.abs(x)
    t = 1.0 / (1.0 + p * ax)
    poly = t * (a1 + t * (a2 + t * (a3 + t * (a4 + t * a5))))
    return s * (1.0 - poly * jnp.exp(-ax * ax))


def _gelu(x):
    return 0.5 * x * (1.0 + _erf_poly(x * jnp.float32(0.7071067811865476)))


def _mlp_body(x_ref, w1t_ref, b1_ref, w2t_ref, b2_ref, o_ref):
    x = x_ref[...] / jnp.float32(float(L))
    h = _gelu(jnp.dot(x, w1t_ref[...], preferred_element_type=jnp.float32)
              + b1_ref[...])
    o_ref[...] = _gelu(jnp.dot(h, w2t_ref[...], preferred_element_type=jnp.float32)
                       + b2_ref[...])


_BM = 4096


def _mlp(pooled_sum, w1t, b1, w2t, b2):
    grid = (B // _BM,)
    return pl.pallas_call(
        _mlp_body,
        grid=grid,
        in_specs=[
            pl.BlockSpec((_BM, HID), lambda i: (i, 0)),
            pl.BlockSpec((HID, HID), lambda i: (0, 0)),
            pl.BlockSpec((1, HID), lambda i: (0, 0)),
            pl.BlockSpec((HID, HID), lambda i: (0, 0)),
            pl.BlockSpec((1, HID), lambda i: (0, 0)),
        ],
        out_specs=pl.BlockSpec((_BM, HID), lambda i: (i, 0)),
        out_shape=jax.ShapeDtypeStruct((B, HID), jnp.float32),
    )(pooled_sum, w1t, b1, w2t, b2)


def kernel(build_order_trace, emb, W1, b1, W2, b2):
    v = build_order_trace.reshape(-1).astype(jnp.int32) + 1
    r = v & (_RC - 1)
    ids_t = (v - r) + ((r & (_QC - 1)) << 2) + (r >> 13)
    table = _relayout(emb.T).reshape(VPAD, PACK)
    pooled_sum = _gather_pool(ids_t, table)
    w1tp = W1.T[_PERM, :]
    return _mlp(pooled_sum, w1tp, b1.reshape(1, HID), W2.T, b2.reshape(1, HID))
